# baseline (speedup 1.0000x reference)
import functools

import jax
import jax.numpy as jnp
from jax import lax
from jax.experimental import pallas as pl
from jax.experimental.pallas import tpu as pltpu
from jax.experimental.pallas import tpu_sc as plsc

F = 128
NC, NS = 2, 16
NW = NC * NS
C = 80


def _silu(x):
    return x * jax.nn.sigmoid(x)


def _dot(x, w):
    return lax.dot_general(x.astype(jnp.bfloat16), w.astype(jnp.bfloat16),
                           (((1,), (0,)), ((), ())),
                           preferred_element_type=jnp.float32)


def _node_tables(h, We1a, We1b, be1):
    N = h.shape[0]
    BN = 2000

    def body(h_ref, wa_ref, wb_ref, b_ref, hs_ref, ht_ref):
        x = h_ref[...]
        hs_ref[...] = _dot(x, wa_ref[...]) + b_ref[...]
        ht_ref[...] = _dot(x, wb_ref[...])

    return pl.pallas_call(
        body,
        grid=(N // BN,),
        in_specs=[
            pl.BlockSpec((BN, F), lambda i: (i, 0)),
            pl.BlockSpec((F, F), lambda i: (0, 0)),
            pl.BlockSpec((F, F), lambda i: (0, 0)),
            pl.BlockSpec((1, F), lambda i: (0, 0)),
        ],
        out_specs=[
            pl.BlockSpec((BN, F), lambda i: (i, 0)),
            pl.BlockSpec((BN, F), lambda i: (i, 0)),
        ],
        out_shape=[
            jax.ShapeDtypeStruct((N, F), jnp.float32),
            jax.ShapeDtypeStruct((N, F), jnp.float32),
        ],
    )(h, We1a, We1b, be1)


def _two_slot_ring(nchunk, issue, wait, process):
    work, drain = process
    issue(0, 0)
    wait(0, 0)
    if nchunk == 1:
        work(0, 0)
        drain(0, 0)
        return
    issue(1, 1)
    work(0, 0)

    if nchunk % 2 == 1:
        @pl.loop(1, nchunk - 2, step=2)
        def _(j):
            wait(j, 1)
            drain(j - 1, 0)
            issue(j + 1, 0)
            work(j, 1)
            wait(j + 1, 0)
            drain(j, 1)
            issue(j + 2, 1)
            work(j + 1, 0)

        j = nchunk - 2
        wait(j, 1)
        drain(j - 1, 0)
        issue(j + 1, 0)
        work(j, 1)
        wait(j + 1, 0)
        drain(j, 1)
        work(j + 1, 0)
        drain(j + 1, 0)
    else:
        @pl.loop(1, nchunk - 1, step=2)
        def _(j):
            wait(j, 1)
            drain(j - 1, 0)
            issue(j + 1, 0)
            work(j, 1)
            wait(j + 1, 0)
            drain(j, 1)
            issue(j + 2, 1)
            work(j + 1, 0)

        j = nchunk - 1
        wait(j, 1)
        drain(j - 1, 0)
        work(j, 1)
        drain(j, 1)


def _gather_add(hs, ht, row, col):
    E = row.shape[0]
    epw = E // NW
    nchunk = epw // C
    mesh = plsc.VectorSubcoreMesh(core_axis_name="c", subcore_axis_name="s")

    @functools.partial(
        pl.kernel,
        mesh=mesh,
        out_type=[jax.ShapeDtypeStruct((E, F), jnp.float32),
                  jax.ShapeDtypeStruct((E, F), jnp.float32)],
        scratch_types=[
            pltpu.VMEM((epw,), jnp.int32),
            pltpu.VMEM((epw,), jnp.int32),
            pltpu.VMEM((C, F), jnp.float32),
            pltpu.VMEM((C, F), jnp.float32),
            pltpu.VMEM((C, F), jnp.float32),
            pltpu.VMEM((C, F), jnp.float32),
            pltpu.SemaphoreType.DMA,
            pltpu.SemaphoreType.DMA,
            pltpu.SemaphoreType.DMA,
            pltpu.SemaphoreType.DMA,
            pltpu.SemaphoreType.DMA,
            pltpu.SemaphoreType.DMA,
        ],
    )
    def k(hs_hbm, ht_hbm, row_hbm, col_hbm, g1_hbm, g2_hbm,
          idxr, idxc, bufa0, bufb0, bufa1, bufb1,
          semga0, semgb0, semga1, semgb1, semw0, semw1):
        wid = lax.axis_index("s") * NC + lax.axis_index("c")
        base = wid * epw
        pltpu.sync_copy(row_hbm.at[pl.ds(base, epw)], idxr)
        pltpu.sync_copy(col_hbm.at[pl.ds(base, epw)], idxc)

        bufa = (bufa0, bufa1)
        bufb = (bufb0, bufb1)
        semga = (semga0, semga1)
        semgb = (semgb0, semgb1)
        semw = (semw0, semw1)

        def gather_descs(j, s):
            da = pltpu.make_async_copy(
                hs_hbm.at[idxr.at[pl.ds(j * C, C)]], bufa[s], semga[s])
            db = pltpu.make_async_copy(
                ht_hbm.at[idxc.at[pl.ds(j * C, C)]], bufb[s], semgb[s])
            return da, db

        def issue(j, s):
            da, db = gather_descs(j, s)
            da.start()
            db.start()

        def wait(j, s):
            da, db = gather_descs(j, s)
            da.wait()
            db.wait()

        def write_descs(j, s):
            d1 = pltpu.make_async_copy(
                bufa[s], g1_hbm.at[pl.ds(base + j * C, C)], semw[s])
            d2 = pltpu.make_async_copy(
                bufb[s], g2_hbm.at[pl.ds(base + j * C, C)], semw[s])
            return d1, d2

        def work(j, s):
            d1, d2 = write_descs(j, s)
            d1.start()
            d2.start()

        def drain(j, s):
            d1, d2 = write_descs(j, s)
            d1.wait()
            d2.wait()

        _two_slot_ring(nchunk, issue, wait, (work, drain))

    return k(hs, ht, row, col)


def _edge_mlp(g1, g2, dist, wd, We2, be2):
    E = g1.shape[0]
    BE = 2560
    assert E % BE == 0

    def body(g1_ref, g2_ref, d_ref, wd_ref, w2_ref, b2_ref, m_ref):
        x = _silu(g1_ref[...] + g2_ref[...] + d_ref[...] * wd_ref[...])
        m_ref[...] = _silu(_dot(x, w2_ref[...]) + b2_ref[...])

    return pl.pallas_call(
        body,
        grid=(E // BE,),
        in_specs=[
            pl.BlockSpec((BE, F), lambda i: (i, 0)),
            pl.BlockSpec((BE, F), lambda i: (i, 0)),
            pl.BlockSpec((BE, 1), lambda i: (i, 0)),
            pl.BlockSpec((1, F), lambda i: (0, 0)),
            pl.BlockSpec((F, F), lambda i: (0, 0)),
            pl.BlockSpec((1, F), lambda i: (0, 0)),
        ],
        out_specs=pl.BlockSpec((BE, F), lambda i: (i, 0)),
        out_shape=jax.ShapeDtypeStruct((E, F), jnp.float32),
    )(g1, g2, dist, wd, We2, be2)


def _segment_sum(m, row2d, N):
    NWq, nchunk, Cq = row2d.shape
    E = NWq * nchunk * Cq
    epw = E // NW
    sl_rows = 624
    tail_extra = N - NS * sl_rows
    mesh = plsc.VectorSubcoreMesh(core_axis_name="c", subcore_axis_name="s")

    @functools.partial(
        pl.kernel,
        mesh=mesh,
        out_type=jax.ShapeDtypeStruct((NC * N, F), jnp.float32),
        scratch_types=[
            pltpu.VMEM((nchunk, Cq), jnp.int32),
            pltpu.VMEM((Cq, F), jnp.float32),
            pltpu.VMEM((Cq, F), jnp.float32),
            pltpu.VMEM_SHARED((N, F), jnp.float32),
            pltpu.SemaphoreType.DMA,
            pltpu.SemaphoreType.DMA,
            pltpu.SemaphoreType.DMA,
            pltpu.SemaphoreType.DMA,
        ],
    )
    def k(m_hbm, row_hbm, z_hbm, out_hbm, idx2d, mbuf0, mbuf1,
          agg_sh, seml0, seml1, sema0, sema1):
        c = lax.axis_index("c")
        s = lax.axis_index("s")
        wid = s * NC + c
        base = wid * epw

        pltpu.sync_copy(row_hbm.at[wid], idx2d)

        pltpu.sync_copy(z_hbm.at[pl.ds(s * sl_rows, sl_rows)],
                        agg_sh.at[pl.ds(s * sl_rows, sl_rows)])

        @pl.when(s == NS - 1)
        def _():
            pltpu.sync_copy(
                z_hbm.at[pl.ds(NS * sl_rows, tail_extra)],
                agg_sh.at[pl.ds(NS * sl_rows, tail_extra)])

        plsc.subcore_barrier()

        mbuf = (mbuf0, mbuf1)
        seml = (seml0, seml1)
        sema = (sema0, sema1)

        def load_desc(j, sl):
            return pltpu.make_async_copy(
                m_hbm.at[pl.ds(base + j * Cq, Cq)], mbuf[sl], seml[sl])

        def add_desc(j, sl):
            return pltpu.make_async_copy(
                mbuf[sl], agg_sh.at[idx2d.at[j]], sema[sl])

        def issue(j, sl):
            load_desc(j, sl).start()

        def wait(j, sl):
            load_desc(j, sl).wait()

        def work(j, sl):
            pltpu.async_copy(mbuf[sl], agg_sh.at[idx2d.at[j]], sema[sl],
                             add=True)

        def drain(j, sl):
            add_desc(j, sl).wait()

        _two_slot_ring(nchunk, issue, wait, (work, drain))

        plsc.subcore_barrier()

        pltpu.sync_copy(agg_sh.at[pl.ds(s * sl_rows, sl_rows)],
                        out_hbm.at[pl.ds(c * N + s * sl_rows, sl_rows)])

        @pl.when(s == NS - 1)
        def _():
            pltpu.sync_copy(
                agg_sh.at[pl.ds(NS * sl_rows, tail_extra)],
                out_hbm.at[pl.ds(c * N + NS * sl_rows, tail_extra)])

    return k(m, row2d, jnp.zeros((N, F), jnp.float32))


def _node_mlp(h, parts_a, parts_b, Wn1a, Wn1b, bn1, Wn2, bn2):
    N = h.shape[0]
    BN = 2000
    nb = N // BN

    def body(h_ref, a0_ref, a1_ref, b0_ref, b1_ref, w1a_ref, w1b_ref,
             b1w_ref, w2_ref, b2_ref, o_ref):
        agg = (a0_ref[...] + a1_ref[...]) + (b0_ref[...] + b1_ref[...])
        x = _silu(_dot(h_ref[...], w1a_ref[...]) + _dot(agg, w1b_ref[...])
                  + b1w_ref[...])
        o_ref[...] = _dot(x, w2_ref[...]) + b2_ref[...]

    return pl.pallas_call(
        body,
        grid=(nb,),
        in_specs=[
            pl.BlockSpec((BN, F), lambda i: (i, 0)),
            pl.BlockSpec((BN, F), lambda i: (i, 0)),
            pl.BlockSpec((BN, F), lambda i: (i + nb, 0)),
            pl.BlockSpec((BN, F), lambda i: (i, 0)),
            pl.BlockSpec((BN, F), lambda i: (i + nb, 0)),
            pl.BlockSpec((F, F), lambda i: (0, 0)),
            pl.BlockSpec((F, F), lambda i: (0, 0)),
            pl.BlockSpec((1, F), lambda i: (0, 0)),
            pl.BlockSpec((F, F), lambda i: (0, 0)),
            pl.BlockSpec((1, F), lambda i: (0, 0)),
        ],
        out_specs=pl.BlockSpec((BN, F), lambda i: (i, 0)),
        out_shape=jax.ShapeDtypeStruct((N, F), jnp.float32),
    )(h, parts_a, parts_a, parts_b, parts_b, Wn1a, Wn1b, bn1, Wn2, bn2)


def kernel(h, row, col, dist, W_e1, b_e1, W_e2, b_e2, W_n1, b_n1, W_n2, b_n2):
    N = h.shape[0]
    E = row.shape[0]
    row = row.astype(jnp.int32)
    col = col.astype(jnp.int32)
    We1a = W_e1[:F]
    We1b = W_e1[F:2 * F]
    wd = W_e1[2 * F:2 * F + 1]
    be1 = b_e1.reshape(1, F)
    be2 = b_e2.reshape(1, F)
    Wn1a = W_n1[:F]
    Wn1b = W_n1[F:]
    bn1 = b_n1.reshape(1, F)
    bn2 = b_n2.reshape(1, F)

    nca = 64
    EA = NW * nca * C
    ra, rb = row[:EA], row[EA:]
    ca_, cb_ = col[:EA], col[EA:]
    da_, db_ = dist[:EA], dist[EA:]
    ra2d = ra.reshape(NW, nca, C)
    rb2d = rb.reshape(NW, (E - EA) // (NW * C), C)

    hs, ht = _node_tables(h, We1a, We1b, be1)

    g1_a, g2_a = _gather_add(hs, ht, ra, ca_)
    g1_b, g2_b = _gather_add(hs, ht, rb, cb_)
    m_a = _edge_mlp(g1_a, g2_a, da_, wd, W_e2, be2)
    m_b = _edge_mlp(g1_b, g2_b, db_, wd, W_e2, be2)
    parts_a = _segment_sum(m_a, ra2d, N)
    parts_b = _segment_sum(m_b, rb2d, N)
    return _node_mlp(h, parts_a, parts_b, Wn1a, Wn1b, bn1, W_n2, bn2)


# speedup vs baseline: 1.1763x; 1.1763x over previous
import dataclasses
import functools

import jax
import jax.numpy as jnp
from jax import lax
from jax.experimental import pallas as pl
from jax.experimental.pallas import tpu as pltpu
from jax.experimental.pallas import tpu_sc as plsc

F = 128
NC, NS = 2, 16
NW = NC * NS
C = 80


def _silu(x):
    return x * jax.nn.sigmoid(x)


def _dot(x, w):
    return lax.dot_general(x.astype(jnp.bfloat16), w.astype(jnp.bfloat16),
                           (((1,), (0,)), ((), ())),
                           preferred_element_type=jnp.float32)


def _node_tables(h, We1a, We1b, be1):
    N = h.shape[0]
    BN = 2000

    def body(h_ref, wa_ref, wb_ref, b_ref, hs_ref, ht_ref):
        x = h_ref[...]
        hs_ref[...] = _dot(x, wa_ref[...]) + b_ref[...]
        ht_ref[...] = _dot(x, wb_ref[...])

    return pl.pallas_call(
        body,
        grid=(N // BN,),
        in_specs=[
            pl.BlockSpec((BN, F), lambda i: (i, 0)),
            pl.BlockSpec((F, F), lambda i: (0, 0)),
            pl.BlockSpec((F, F), lambda i: (0, 0)),
            pl.BlockSpec((1, F), lambda i: (0, 0)),
        ],
        out_specs=[
            pl.BlockSpec((BN, F), lambda i: (i, 0)),
            pl.BlockSpec((BN, F), lambda i: (i, 0)),
        ],
        out_shape=[
            jax.ShapeDtypeStruct((N, F), jnp.float32),
            jax.ShapeDtypeStruct((N, F), jnp.float32),
        ],
    )(h, We1a, We1b, be1)


def _two_slot_ring(nchunk, issue, wait, process):
    work, drain = process
    issue(0, 0)
    wait(0, 0)
    if nchunk == 1:
        work(0, 0)
        drain(0, 0)
        return
    issue(1, 1)
    work(0, 0)

    if nchunk % 2 == 1:
        @pl.loop(1, nchunk - 2, step=2)
        def _(j):
            wait(j, 1)
            drain(j - 1, 0)
            issue(j + 1, 0)
            work(j, 1)
            wait(j + 1, 0)
            drain(j, 1)
            issue(j + 2, 1)
            work(j + 1, 0)

        j = nchunk - 2
        wait(j, 1)
        drain(j - 1, 0)
        issue(j + 1, 0)
        work(j, 1)
        wait(j + 1, 0)
        drain(j, 1)
        work(j + 1, 0)
        drain(j + 1, 0)
    else:
        @pl.loop(1, nchunk - 1, step=2)
        def _(j):
            wait(j, 1)
            drain(j - 1, 0)
            issue(j + 1, 0)
            work(j, 1)
            wait(j + 1, 0)
            drain(j, 1)
            issue(j + 2, 1)
            work(j + 1, 0)

        j = nchunk - 1
        wait(j, 1)
        drain(j - 1, 0)
        work(j, 1)
        drain(j, 1)


def _gather_add(hs, ht, row, col):
    E = row.shape[0]
    epw = E // NW
    nchunk = epw // C
    assert nchunk >= 4
    mesh = plsc.VectorSubcoreMesh(core_axis_name="c", subcore_axis_name="s")

    @functools.partial(
        pl.kernel,
        mesh=mesh,
        out_type=jax.ShapeDtypeStruct((E, F), jnp.float32),
        scratch_types=(
            [pltpu.VMEM((epw,), jnp.int32)] * 2
            + [pltpu.VMEM((C, F), jnp.float32)] * 6
            + [pltpu.SemaphoreType.DMA] * 9
        ),
    )
    def k(hs_hbm, ht_hbm, row_hbm, col_hbm, g_hbm,
          idxr, idxc, bufa0, bufb0, bufa1, bufb1, bufa2, bufb2,
          semga0, semgb0, semga1, semgb1, semga2, semgb2,
          semw0, semw1, semw2):
        wid = lax.axis_index("s") * NC + lax.axis_index("c")
        base = wid * epw
        pltpu.sync_copy(row_hbm.at[pl.ds(base, epw)], idxr)
        pltpu.sync_copy(col_hbm.at[pl.ds(base, epw)], idxc)

        bufa = (bufa0, bufa1, bufa2)
        bufb = (bufb0, bufb1, bufb2)
        semga = (semga0, semga1, semga2)
        semgb = (semgb0, semgb1, semgb2)
        semw = (semw0, semw1, semw2)

        def gather_descs(j, s):
            da = pltpu.make_async_copy(
                hs_hbm.at[idxr.at[pl.ds(j * C, C)]], bufa[s], semga[s])
            db = pltpu.make_async_copy(
                ht_hbm.at[idxc.at[pl.ds(j * C, C)]], bufb[s], semgb[s])
            return da, db

        def issue(j, s):
            da, db = gather_descs(j, s)
            da.start()
            db.start()

        def wait(j, s):
            da, db = gather_descs(j, s)
            da.wait()
            db.wait()

        def write_desc(j, s):
            return pltpu.make_async_copy(
                bufa[s], g_hbm.at[pl.ds(base + j * C, C)], semw[s])

        def add_write(j, s):
            a, b = bufa[s], bufb[s]

            @pl.loop(0, C)
            def _(r):
                for t in range(F // 16):
                    sl = pl.ds(t * 16, 16)
                    plsc.addupdate(a.at[r, sl], b[r, sl])

            write_desc(j, s).start()

        def steady(j, s):
            s1 = (s + 1) % 3
            wait(j, s)
            write_desc(j - 2, s1).wait()
            issue(j + 1, s1)
            add_write(j, s)

        issue(0, 0)
        wait(0, 0)
        issue(1, 1)
        add_write(0, 0)
        wait(1, 1)
        issue(2, 2)
        add_write(1, 1)

        n_steady = nchunk - 3
        k3 = n_steady // 3
        rem = n_steady % 3

        @pl.loop(2, 2 + 3 * k3, step=3)
        def _(j):
            steady(j, 2)
            steady(j + 1, 0)
            steady(j + 2, 1)

        for r in range(rem):
            steady(2 + 3 * k3 + r, (2 + r) % 3)

        jf = nchunk - 1
        wait(jf, jf % 3)
        add_write(jf, jf % 3)
        write_desc(jf - 2, (jf - 2) % 3).wait()
        write_desc(jf - 1, (jf - 1) % 3).wait()
        write_desc(jf, jf % 3).wait()

    return k(hs, ht, row, col)


def _edge_mlp(g, dist, wd, We2, be2):
    E = g.shape[0]
    BE = 2560
    assert E % BE == 0

    def body(g_ref, d_ref, wd_ref, w2_ref, b2_ref, m_ref):
        x = _silu(g_ref[...].astype(jnp.float32) + d_ref[...] * wd_ref[...])
        m_ref[...] = _silu(_dot(x, w2_ref[...]) + b2_ref[...])

    return pl.pallas_call(
        body,
        grid=(E // BE,),
        in_specs=[
            pl.BlockSpec((BE, F), lambda i: (i, 0)),
            pl.BlockSpec((BE, 1), lambda i: (i, 0)),
            pl.BlockSpec((1, F), lambda i: (0, 0)),
            pl.BlockSpec((F, F), lambda i: (0, 0)),
            pl.BlockSpec((1, F), lambda i: (0, 0)),
        ],
        out_specs=pl.BlockSpec((BE, F), lambda i: (i, 0)),
        out_shape=jax.ShapeDtypeStruct((E, F), jnp.float32),
    )(g, dist, wd, We2, be2)


def _segment_sum(m, row2d, N):
    NWq, nchunk, Cq = row2d.shape
    E = NWq * nchunk * Cq
    epw = E // NW
    sl_rows = 624
    tail_extra = N - NS * sl_rows
    mesh = plsc.VectorSubcoreMesh(core_axis_name="c", subcore_axis_name="s")

    @functools.partial(
        pl.kernel,
        mesh=mesh,
        out_type=jax.ShapeDtypeStruct((NC * N, F), jnp.float32),
        scratch_types=[
            pltpu.VMEM((nchunk, Cq), jnp.int32),
            pltpu.VMEM((Cq, F), jnp.float32),
            pltpu.VMEM((Cq, F), jnp.float32),
            pltpu.VMEM_SHARED((N, F), jnp.float32),
            pltpu.SemaphoreType.DMA,
            pltpu.SemaphoreType.DMA,
            pltpu.SemaphoreType.DMA,
            pltpu.SemaphoreType.DMA,
        ],
    )
    def k(m_hbm, row_hbm, z_hbm, out_hbm, idx2d, mbuf0, mbuf1,
          agg_sh, seml0, seml1, sema0, sema1):
        c = lax.axis_index("c")
        s = lax.axis_index("s")
        wid = s * NC + c
        base = wid * epw

        pltpu.sync_copy(row_hbm.at[wid], idx2d)

        pltpu.sync_copy(z_hbm.at[pl.ds(s * sl_rows, sl_rows)],
                        agg_sh.at[pl.ds(s * sl_rows, sl_rows)])

        @pl.when(s == NS - 1)
        def _():
            pltpu.sync_copy(
                z_hbm.at[pl.ds(NS * sl_rows, tail_extra)],
                agg_sh.at[pl.ds(NS * sl_rows, tail_extra)])

        plsc.subcore_barrier()

        mbuf = (mbuf0, mbuf1)
        seml = (seml0, seml1)
        sema = (sema0, sema1)

        def load_desc(j, sl):
            return pltpu.make_async_copy(
                m_hbm.at[pl.ds(base + j * Cq, Cq)], mbuf[sl], seml[sl])

        def add_desc(j, sl):
            return pltpu.make_async_copy(
                mbuf[sl], agg_sh.at[idx2d.at[j]], sema[sl])

        def issue(j, sl):
            load_desc(j, sl).start()

        def wait(j, sl):
            load_desc(j, sl).wait()

        def work(j, sl):
            pltpu.async_copy(mbuf[sl], agg_sh.at[idx2d.at[j]], sema[sl],
                             add=True)

        def drain(j, sl):
            add_desc(j, sl).wait()

        _two_slot_ring(nchunk, issue, wait, (work, drain))

        plsc.subcore_barrier()

        pltpu.sync_copy(agg_sh.at[pl.ds(s * sl_rows, sl_rows)],
                        out_hbm.at[pl.ds(c * N + s * sl_rows, sl_rows)])

        @pl.when(s == NS - 1)
        def _():
            pltpu.sync_copy(
                agg_sh.at[pl.ds(NS * sl_rows, tail_extra)],
                out_hbm.at[pl.ds(c * N + NS * sl_rows, tail_extra)])

    return k(m, row2d, jnp.zeros((N, F), jnp.float32))


def _node_mlp(h, parts_a, parts_b, Wn1a, Wn1b, bn1, Wn2, bn2):
    N = h.shape[0]
    BN = 2000
    nb = N // BN

    def body(h_ref, a0_ref, a1_ref, b0_ref, b1_ref, w1a_ref, w1b_ref,
             b1w_ref, w2_ref, b2_ref, o_ref):
        agg = (a0_ref[...] + a1_ref[...]) + (b0_ref[...] + b1_ref[...])
        x = _silu(_dot(h_ref[...], w1a_ref[...]) + _dot(agg, w1b_ref[...])
                  + b1w_ref[...])
        o_ref[...] = _dot(x, w2_ref[...]) + b2_ref[...]

    return pl.pallas_call(
        body,
        grid=(nb,),
        in_specs=[
            pl.BlockSpec((BN, F), lambda i: (i, 0)),
            pl.BlockSpec((BN, F), lambda i: (i, 0)),
            pl.BlockSpec((BN, F), lambda i: (i + nb, 0)),
            pl.BlockSpec((BN, F), lambda i: (i, 0)),
            pl.BlockSpec((BN, F), lambda i: (i + nb, 0)),
            pl.BlockSpec((F, F), lambda i: (0, 0)),
            pl.BlockSpec((F, F), lambda i: (0, 0)),
            pl.BlockSpec((1, F), lambda i: (0, 0)),
            pl.BlockSpec((F, F), lambda i: (0, 0)),
            pl.BlockSpec((1, F), lambda i: (0, 0)),
        ],
        out_specs=pl.BlockSpec((BN, F), lambda i: (i, 0)),
        out_shape=jax.ShapeDtypeStruct((N, F), jnp.float32),
    )(h, parts_a, parts_a, parts_b, parts_b, Wn1a, Wn1b, bn1, Wn2, bn2)


def kernel(h, row, col, dist, W_e1, b_e1, W_e2, b_e2, W_n1, b_n1, W_n2, b_n2):
    N = h.shape[0]
    E = row.shape[0]
    row = row.astype(jnp.int32)
    col = col.astype(jnp.int32)
    We1a = W_e1[:F]
    We1b = W_e1[F:2 * F]
    wd = W_e1[2 * F:2 * F + 1]
    be1 = b_e1.reshape(1, F)
    be2 = b_e2.reshape(1, F)
    Wn1a = W_n1[:F]
    Wn1b = W_n1[F:]
    bn1 = b_n1.reshape(1, F)
    bn2 = b_n2.reshape(1, F)

    nca = 64
    EA = NW * nca * C
    ra, rb = row[:EA], row[EA:]
    ca_, cb_ = col[:EA], col[EA:]
    da_, db_ = dist[:EA], dist[EA:]
    ra2d = ra.reshape(NW, nca, C)
    rb2d = rb.reshape(NW, (E - EA) // (NW * C), C)

    hs, ht = _node_tables(h, We1a, We1b, be1)

    g_a = _gather_add(hs, ht, ra, ca_)
    g_b = _gather_add(hs, ht, rb, cb_)
    m_a = _edge_mlp(g_a, da_, wd, W_e2, be2)
    m_b = _edge_mlp(g_b, db_, wd, W_e2, be2)
    parts_a = _segment_sum(m_a, ra2d, N)
    parts_b = _segment_sum(m_b, rb2d, N)
    return _node_mlp(h, parts_a, parts_b, Wn1a, Wn1b, bn1, W_n2, bn2)


# speedup vs baseline: 1.1915x; 1.0129x over previous
import dataclasses
import functools

import jax
import jax.numpy as jnp
from jax import lax
from jax.experimental import pallas as pl
from jax.experimental.pallas import tpu as pltpu
from jax.experimental.pallas import tpu_sc as plsc

F = 128
NC, NS = 2, 16
NW = NC * NS
C = 80


def _silu(x):
    return x * jax.nn.sigmoid(x)


def _dot(x, w):
    return lax.dot_general(x.astype(jnp.bfloat16), w.astype(jnp.bfloat16),
                           (((1,), (0,)), ((), ())),
                           preferred_element_type=jnp.float32)


def _node_tables(h, We1a, We1b, be1):
    N = h.shape[0]
    BN = 2000

    def body(h_ref, wa_ref, wb_ref, b_ref, hs_ref, ht_ref):
        x = h_ref[...]
        hs_ref[...] = _dot(x, wa_ref[...]) + b_ref[...]
        ht_ref[...] = _dot(x, wb_ref[...])

    return pl.pallas_call(
        body,
        grid=(N // BN,),
        in_specs=[
            pl.BlockSpec((BN, F), lambda i: (i, 0)),
            pl.BlockSpec((F, F), lambda i: (0, 0)),
            pl.BlockSpec((F, F), lambda i: (0, 0)),
            pl.BlockSpec((1, F), lambda i: (0, 0)),
        ],
        out_specs=[
            pl.BlockSpec((BN, F), lambda i: (i, 0)),
            pl.BlockSpec((BN, F), lambda i: (i, 0)),
        ],
        out_shape=[
            jax.ShapeDtypeStruct((N, F), jnp.float32),
            jax.ShapeDtypeStruct((N, F), jnp.float32),
        ],
    )(h, We1a, We1b, be1)


def _two_slot_ring(nchunk, issue, wait, process):
    work, drain = process
    issue(0, 0)
    wait(0, 0)
    if nchunk == 1:
        work(0, 0)
        drain(0, 0)
        return
    issue(1, 1)
    work(0, 0)

    if nchunk % 2 == 1:
        @pl.loop(1, nchunk - 2, step=2)
        def _(j):
            wait(j, 1)
            drain(j - 1, 0)
            issue(j + 1, 0)
            work(j, 1)
            wait(j + 1, 0)
            drain(j, 1)
            issue(j + 2, 1)
            work(j + 1, 0)

        j = nchunk - 2
        wait(j, 1)
        drain(j - 1, 0)
        issue(j + 1, 0)
        work(j, 1)
        wait(j + 1, 0)
        drain(j, 1)
        work(j + 1, 0)
        drain(j + 1, 0)
    else:
        @pl.loop(1, nchunk - 1, step=2)
        def _(j):
            wait(j, 1)
            drain(j - 1, 0)
            issue(j + 1, 0)
            work(j, 1)
            wait(j + 1, 0)
            drain(j, 1)
            issue(j + 2, 1)
            work(j + 1, 0)

        j = nchunk - 1
        wait(j, 1)
        drain(j - 1, 0)
        work(j, 1)
        drain(j, 1)


def _gather_add(hs, ht, row, col, C=C):
    E = row.shape[0]
    epw = E // NW
    nchunk = epw // C
    assert nchunk * C == epw and nchunk >= 4
    mesh = plsc.VectorSubcoreMesh(core_axis_name="c", subcore_axis_name="s")

    @functools.partial(
        pl.kernel,
        mesh=mesh,
        out_type=jax.ShapeDtypeStruct((E, F), jnp.float32),
        scratch_types=(
            [pltpu.VMEM((epw,), jnp.int32)] * 2
            + [pltpu.VMEM((C, F), jnp.float32)] * 6
            + [pltpu.SemaphoreType.DMA] * 9
        ),
    )
    def k(hs_hbm, ht_hbm, row_hbm, col_hbm, g_hbm,
          idxr, idxc, bufa0, bufb0, bufa1, bufb1, bufa2, bufb2,
          semga0, semgb0, semga1, semgb1, semga2, semgb2,
          semw0, semw1, semw2):
        wid = lax.axis_index("s") * NC + lax.axis_index("c")
        base = wid * epw
        pltpu.sync_copy(row_hbm.at[pl.ds(base, epw)], idxr)
        pltpu.sync_copy(col_hbm.at[pl.ds(base, epw)], idxc)

        bufa = (bufa0, bufa1, bufa2)
        bufb = (bufb0, bufb1, bufb2)
        semga = (semga0, semga1, semga2)
        semgb = (semgb0, semgb1, semgb2)
        semw = (semw0, semw1, semw2)

        def gather_descs(j, s):
            da = pltpu.make_async_copy(
                hs_hbm.at[idxr.at[pl.ds(j * C, C)]], bufa[s], semga[s])
            db = pltpu.make_async_copy(
                ht_hbm.at[idxc.at[pl.ds(j * C, C)]], bufb[s], semgb[s])
            return da, db

        def issue(j, s):
            da, db = gather_descs(j, s)
            da.start()
            db.start()

        def wait(j, s):
            da, db = gather_descs(j, s)
            da.wait()
            db.wait()

        def write_desc(j, s):
            return pltpu.make_async_copy(
                bufa[s], g_hbm.at[pl.ds(base + j * C, C)], semw[s])

        def add_write(j, s):
            a, b = bufa[s], bufb[s]

            @pl.loop(0, C)
            def _(r):
                for t in range(F // 16):
                    sl = pl.ds(t * 16, 16)
                    plsc.addupdate(a.at[r, sl], b[r, sl])

            write_desc(j, s).start()

        def steady(j, s):
            s1 = (s + 1) % 3
            wait(j, s)
            write_desc(j - 2, s1).wait()
            issue(j + 1, s1)
            add_write(j, s)

        issue(0, 0)
        wait(0, 0)
        issue(1, 1)
        add_write(0, 0)
        wait(1, 1)
        issue(2, 2)
        add_write(1, 1)

        n_steady = nchunk - 3
        k3 = n_steady // 3
        rem = n_steady % 3

        @pl.loop(2, 2 + 3 * k3, step=3)
        def _(j):
            steady(j, 2)
            steady(j + 1, 0)
            steady(j + 2, 1)

        for r in range(rem):
            steady(2 + 3 * k3 + r, (2 + r) % 3)

        jf = nchunk - 1
        wait(jf, jf % 3)
        add_write(jf, jf % 3)
        write_desc(jf - 2, (jf - 2) % 3).wait()
        write_desc(jf - 1, (jf - 1) % 3).wait()
        write_desc(jf, jf % 3).wait()

    return k(hs, ht, row, col)


def _edge_mlp(g, dist, wd, We2, be2):
    E = g.shape[0]
    BE = 2560
    assert E % BE == 0

    def body(g_ref, d_ref, wd_ref, w2_ref, b2_ref, m_ref):
        x = _silu(g_ref[...].astype(jnp.float32) + d_ref[...] * wd_ref[...])
        m_ref[...] = _silu(_dot(x, w2_ref[...]) + b2_ref[...])

    return pl.pallas_call(
        body,
        grid=(E // BE,),
        in_specs=[
            pl.BlockSpec((BE, F), lambda i: (i, 0)),
            pl.BlockSpec((BE, 1), lambda i: (i, 0)),
            pl.BlockSpec((1, F), lambda i: (0, 0)),
            pl.BlockSpec((F, F), lambda i: (0, 0)),
            pl.BlockSpec((1, F), lambda i: (0, 0)),
        ],
        out_specs=pl.BlockSpec((BE, F), lambda i: (i, 0)),
        out_shape=jax.ShapeDtypeStruct((E, F), jnp.float32),
    )(g, dist, wd, We2, be2)


def _segment_sum(m, row2d, N):
    NWq, nchunk, Cq = row2d.shape
    E = NWq * nchunk * Cq
    epw = E // NW
    sl_rows = 624
    tail_extra = N - NS * sl_rows
    mesh = plsc.VectorSubcoreMesh(core_axis_name="c", subcore_axis_name="s")

    @functools.partial(
        pl.kernel,
        mesh=mesh,
        out_type=jax.ShapeDtypeStruct((NC * N, F), jnp.float32),
        scratch_types=[
            pltpu.VMEM((nchunk, Cq), jnp.int32),
            pltpu.VMEM((Cq, F), jnp.float32),
            pltpu.VMEM((Cq, F), jnp.float32),
            pltpu.VMEM_SHARED((N, F), jnp.float32),
            pltpu.SemaphoreType.DMA,
            pltpu.SemaphoreType.DMA,
            pltpu.SemaphoreType.DMA,
            pltpu.SemaphoreType.DMA,
        ],
    )
    def k(m_hbm, row_hbm, z_hbm, out_hbm, idx2d, mbuf0, mbuf1,
          agg_sh, seml0, seml1, sema0, sema1):
        c = lax.axis_index("c")
        s = lax.axis_index("s")
        wid = s * NC + c
        base = wid * epw

        pltpu.sync_copy(row_hbm.at[wid], idx2d)

        pltpu.sync_copy(z_hbm.at[pl.ds(s * sl_rows, sl_rows)],
                        agg_sh.at[pl.ds(s * sl_rows, sl_rows)])

        @pl.when(s == NS - 1)
        def _():
            pltpu.sync_copy(
                z_hbm.at[pl.ds(NS * sl_rows, tail_extra)],
                agg_sh.at[pl.ds(NS * sl_rows, tail_extra)])

        plsc.subcore_barrier()

        mbuf = (mbuf0, mbuf1)
        seml = (seml0, seml1)
        sema = (sema0, sema1)

        def load_desc(j, sl):
            return pltpu.make_async_copy(
                m_hbm.at[pl.ds(base + j * Cq, Cq)], mbuf[sl], seml[sl])

        def add_desc(j, sl):
            return pltpu.make_async_copy(
                mbuf[sl], agg_sh.at[idx2d.at[j]], sema[sl])

        def issue(j, sl):
            load_desc(j, sl).start()

        def wait(j, sl):
            load_desc(j, sl).wait()

        def work(j, sl):
            pltpu.async_copy(mbuf[sl], agg_sh.at[idx2d.at[j]], sema[sl],
                             add=True)

        def drain(j, sl):
            add_desc(j, sl).wait()

        _two_slot_ring(nchunk, issue, wait, (work, drain))

        plsc.subcore_barrier()

        pltpu.sync_copy(agg_sh.at[pl.ds(s * sl_rows, sl_rows)],
                        out_hbm.at[pl.ds(c * N + s * sl_rows, sl_rows)])

        @pl.when(s == NS - 1)
        def _():
            pltpu.sync_copy(
                agg_sh.at[pl.ds(NS * sl_rows, tail_extra)],
                out_hbm.at[pl.ds(c * N + NS * sl_rows, tail_extra)])

    return k(m, row2d, jnp.zeros((N, F), jnp.float32))


def _node_mlp(h, parts_a, parts_b, Wn1a, Wn1b, bn1, Wn2, bn2):
    N = h.shape[0]
    BN = 2000
    nb = N // BN

    def body(h_ref, a0_ref, a1_ref, b0_ref, b1_ref, w1a_ref, w1b_ref,
             b1w_ref, w2_ref, b2_ref, o_ref):
        agg = (a0_ref[...] + a1_ref[...]) + (b0_ref[...] + b1_ref[...])
        x = _silu(_dot(h_ref[...], w1a_ref[...]) + _dot(agg, w1b_ref[...])
                  + b1w_ref[...])
        o_ref[...] = _dot(x, w2_ref[...]) + b2_ref[...]

    return pl.pallas_call(
        body,
        grid=(nb,),
        in_specs=[
            pl.BlockSpec((BN, F), lambda i: (i, 0)),
            pl.BlockSpec((BN, F), lambda i: (i, 0)),
            pl.BlockSpec((BN, F), lambda i: (i + nb, 0)),
            pl.BlockSpec((BN, F), lambda i: (i, 0)),
            pl.BlockSpec((BN, F), lambda i: (i + nb, 0)),
            pl.BlockSpec((F, F), lambda i: (0, 0)),
            pl.BlockSpec((F, F), lambda i: (0, 0)),
            pl.BlockSpec((1, F), lambda i: (0, 0)),
            pl.BlockSpec((F, F), lambda i: (0, 0)),
            pl.BlockSpec((1, F), lambda i: (0, 0)),
        ],
        out_specs=pl.BlockSpec((BN, F), lambda i: (i, 0)),
        out_shape=jax.ShapeDtypeStruct((N, F), jnp.float32),
    )(h, parts_a, parts_a, parts_b, parts_b, Wn1a, Wn1b, bn1, Wn2, bn2)


def kernel(h, row, col, dist, W_e1, b_e1, W_e2, b_e2, W_n1, b_n1, W_n2, b_n2):
    N = h.shape[0]
    E = row.shape[0]
    row = row.astype(jnp.int32)
    col = col.astype(jnp.int32)
    We1a = W_e1[:F]
    We1b = W_e1[F:2 * F]
    wd = W_e1[2 * F:2 * F + 1]
    be1 = b_e1.reshape(1, F)
    be2 = b_e2.reshape(1, F)
    Wn1a = W_n1[:F]
    Wn1b = W_n1[F:]
    bn1 = b_n1.reshape(1, F)
    bn2 = b_n2.reshape(1, F)

    nca = 64
    EA = NW * nca * C
    ra, rb = row[:EA], row[EA:]
    ca_, cb_ = col[:EA], col[EA:]
    da_, db_ = dist[:EA], dist[EA:]
    ra2d = ra.reshape(NW, nca, C)
    rb2d = rb.reshape(NW, (E - EA) // (NW * C), C)

    hs, ht = _node_tables(h, We1a, We1b, be1)

    g_a = _gather_add(hs, ht, ra, ca_, C=128)
    g_b = _gather_add(hs, ht, rb, cb_)
    m_a = _edge_mlp(g_a, da_, wd, W_e2, be2)
    m_b = _edge_mlp(g_b, db_, wd, W_e2, be2)
    parts_a = _segment_sum(m_a, ra2d, N)
    parts_b = _segment_sum(m_b, rb2d, N)
    return _node_mlp(h, parts_a, parts_b, Wn1a, Wn1b, bn1, W_n2, bn2)
